# baseline (speedup 1.0000x reference)
import jax
import jax.numpy as jnp
from jax.experimental import pallas as pl
from jax.experimental.pallas import tpu as pltpu

_IDX0 = 7
_STRIDE = 156
_NPAIRS = 64
_NUM_NODES = 10000
_NBUF = 4
_MULT = 2
_BLK = _MULT * _STRIDE
_NSTEPS = _NPAIRS // _MULT
_LASTBLK = _NUM_NODES - (_NSTEPS - 1) * _BLK
_GROUPS = 8


def _body(x_ref, o_ref, e2g, scr, sem):
    i = pl.program_id(0)
    b = jax.lax.rem(i, _NBUF)

    @pl.when(i == 0)
    def _():
        x = x_ref[...]
        for g in range(_GROUPS):
            xg = jax.lax.slice(x, (g * 128, 0), ((g + 1) * 128, 128))
            e2g[g] = jnp.swapaxes(xg, 0, 1)

    @pl.when(i < _NBUF)
    def _():
        scr[b] = jnp.zeros((_LASTBLK, 2 * _GROUPS, 128), jnp.float32)

    @pl.when(i >= _NBUF)
    def _():
        pltpu.make_async_copy(
            scr.at[b, :_BLK], o_ref.at[pl.ds((i - _NBUF) * _BLK, _BLK)],
            sem.at[b]).wait()

    for m in range(_MULT):
        for g in range(_GROUPS):
            for j in range(2):
                scr[b, _IDX0 + m * _STRIDE, 2 * g + j, :] = (
                    e2g[g, 2 * (_MULT * i + m) + j, :])

    @pl.when(i < _NSTEPS - 1)
    def _():
        pltpu.make_async_copy(
            scr.at[b, :_BLK], o_ref.at[pl.ds(i * _BLK, _BLK)],
            sem.at[b]).start()

    @pl.when(i == _NSTEPS - 1)
    def _():
        pltpu.make_async_copy(
            scr.at[b], o_ref.at[pl.ds(i * _BLK, _LASTBLK)],
            sem.at[b]).start()
        for j in range(_NBUF - 1):
            s = _NSTEPS - _NBUF + j
            pltpu.make_async_copy(
                scr.at[s % _NBUF, :_BLK], o_ref.at[pl.ds(s * _BLK, _BLK)],
                sem.at[s % _NBUF]).wait()
        pltpu.make_async_copy(
            scr.at[b], o_ref.at[pl.ds(i * _BLK, _LASTBLK)],
            sem.at[b]).wait()


def kernel(f_lat):
    rows = f_lat.shape[0]
    out = pl.pallas_call(
        _body,
        grid=(_NSTEPS,),
        in_specs=[pl.BlockSpec((rows, 128), lambda i: (0, 0))],
        out_specs=pl.BlockSpec(memory_space=pl.ANY),
        out_shape=jax.ShapeDtypeStruct((_NUM_NODES, 2 * _GROUPS, 128),
                                       f_lat.dtype),
        scratch_shapes=[
            pltpu.VMEM((_GROUPS, 128, 128), jnp.float32),
            pltpu.VMEM((_NBUF, _LASTBLK, 2 * _GROUPS, 128), jnp.float32),
            pltpu.SemaphoreType.DMA((_NBUF,)),
        ],
    )(f_lat)

    return (
        out.reshape(_NUM_NODES, _GROUPS, 2, 128)
        .transpose(1, 3, 0, 2)
        .reshape(rows, _NUM_NODES, 2)
    )


# speedup vs baseline: 1.0636x; 1.0636x over previous
import jax
import jax.numpy as jnp
from jax.experimental import pallas as pl
from jax.experimental.pallas import tpu as pltpu

_IDX0 = 7
_STRIDE = 156
_NPAIRS = 64
_NUM_NODES = 10000
_TAIL = _NUM_NODES - _NPAIRS * _STRIDE
_NBUF = 4
_GROUPS = 8


def _body(x_ref, o_ref, e2g, scr, ztail, sem, zsem):
    i = pl.program_id(0)
    n = pl.num_programs(0)
    b = jax.lax.rem(i, _NBUF)

    @pl.when(i == 0)
    def _():
        ztail[...] = jnp.zeros_like(ztail)
        x = x_ref[...]
        for g in range(_GROUPS):
            xg = jax.lax.slice(x, (g * 128, 0), ((g + 1) * 128, 128))
            e2g[g] = jnp.swapaxes(xg, 0, 1)

    @pl.when(i < _NBUF)
    def _():
        scr[b] = jnp.zeros((_STRIDE, 2 * _GROUPS, 128), jnp.float32)

    @pl.when(i < _NPAIRS)
    def _():
        @pl.when(i >= _NBUF)
        def _():
            pltpu.make_async_copy(
                scr.at[b], o_ref.at[pl.ds((i - _NBUF) * _STRIDE, _STRIDE)],
                sem.at[b]).wait()

        for g in range(_GROUPS):
            scr[b, _IDX0, 2 * g:2 * g + 2, :] = e2g[g, pl.ds(2 * i, 2), :]
        pltpu.make_async_copy(
            scr.at[b], o_ref.at[pl.ds(i * _STRIDE, _STRIDE)],
            sem.at[b]).start()

    @pl.when(i == n - 1)
    def _():
        pltpu.make_async_copy(
            ztail, o_ref.at[pl.ds(_NPAIRS * _STRIDE, _TAIL)], zsem).start()
        for j in range(_NBUF):
            s = _NPAIRS - _NBUF + j
            pltpu.make_async_copy(
                scr.at[s % _NBUF], o_ref.at[pl.ds(s * _STRIDE, _STRIDE)],
                sem.at[s % _NBUF]).wait()
        pltpu.make_async_copy(
            ztail, o_ref.at[pl.ds(_NPAIRS * _STRIDE, _TAIL)], zsem).wait()


def kernel(f_lat):
    rows = f_lat.shape[0]
    out = pl.pallas_call(
        _body,
        grid=(_NPAIRS + 1,),
        in_specs=[pl.BlockSpec((rows, 128), lambda i: (0, 0))],
        out_specs=pl.BlockSpec(memory_space=pl.ANY),
        out_shape=jax.ShapeDtypeStruct((_NUM_NODES, 2 * _GROUPS, 128),
                                       f_lat.dtype),
        scratch_shapes=[
            pltpu.VMEM((_GROUPS, 128, 128), jnp.float32),
            pltpu.VMEM((_NBUF, _STRIDE, 2 * _GROUPS, 128), jnp.float32),
            pltpu.VMEM((_TAIL, 2 * _GROUPS, 128), jnp.float32),
            pltpu.SemaphoreType.DMA((_NBUF,)),
            pltpu.SemaphoreType.DMA,
        ],
    )(f_lat)

    return (
        out.reshape(_NUM_NODES, _GROUPS, 2, 128)
        .transpose(1, 3, 0, 2)
        .reshape(rows, _NUM_NODES, 2)
    )
